# Initial kernel scaffold; baseline (speedup 1.0000x reference)
import jax
import jax.numpy as jnp
from jax.experimental import pallas as pl


def kernel(bar, pos, token, dur, phrase, W_bar, W_pos, W_token, W_dur, W_phrase, W_proj, b_proj):
    raise NotImplementedError("write your pallas kernel here")


# speedup vs baseline: 4.1701x; 4.1701x over previous
import functools

import jax
import jax.numpy as jnp
import numpy as np
from jax import lax
from jax.experimental import pallas as pl
from jax.experimental.pallas import tpu as pltpu
from jax.experimental.pallas import tpu_sc as plsc

B, L, D = 1024, 50, 512
NT = B * L

LPAD = 56
N_T1, N_T2, N_T3 = 16 * 32, 64 * LPAD, 256 * 96

NC, NS = 2, 16
NW = NC * NS
PER_W = NT // NW
T = 32
NCHUNK = PER_W // T
NPAIR = NCHUNK // 2


def _pos_enc(seq_len, d):
    channels = int(np.ceil(d / 2) * 2)
    inv_freq = 1.0 / (10000.0 ** (np.arange(0, channels, 2, dtype=np.float32) / np.float32(channels)))
    pos = jnp.arange(seq_len, dtype=jnp.float32)
    sin_inp = pos[:, None] * jnp.asarray(inv_freq, dtype=jnp.float32)[None, :]
    emb = jnp.reshape(jnp.stack((jnp.sin(sin_inp), jnp.cos(sin_inp)), axis=-1), (seq_len, channels))
    return emb[:, :d]


def _prep_body(wbar, wpos, wtok, wdur, wphr, wproj, bias, pe,
               t1_ref, t2_ref, ptok_ref, pdur_ref):
    dn = (((1,), (1,)), ((), ()))

    def proj(tab, seg):
        w = wproj[:, seg * D:(seg + 1) * D]
        return lax.dot_general(tab[...], w, dn, preferred_element_type=jnp.float32)

    pbar = proj(wbar, 0)
    pphr = proj(wphr, 4)
    t1_ref[...] = pbar[:, None, :] + pphr[None, :, :]

    ppos = proj(wpos, 1)
    peb = jnp.concatenate(
        [pe[...] + bias[...][None, :], jnp.zeros((LPAD - L, D), jnp.float32)], axis=0)
    t2_ref[...] = ppos[:, None, :] + peb[None, :, :]

    ptok_ref[...] = proj(wtok, 2)
    pdur_ref[...] = proj(wdur, 3)


def _prep(wbar, wpos, wtok, wdur, wphr, wproj, bias, pe):
    return pl.pallas_call(
        _prep_body,
        out_shape=(
            jax.ShapeDtypeStruct((16, 32, D), jnp.float32),
            jax.ShapeDtypeStruct((64, LPAD, D), jnp.float32),
            jax.ShapeDtypeStruct((256, D), jnp.float32),
            jax.ShapeDtypeStruct((96, D), jnp.float32),
        ),
    )(wbar, wpos, wtok, wdur, wphr, wproj, bias, pe)


def _t3_body(ptok, pdur, t3_ref):
    t3_ref[...] = ptok[...][:, None, :] + pdur[...][None, :, :]


def _t3(ptok, pdur):
    nblk = 8
    return pl.pallas_call(
        _t3_body,
        grid=(nblk,),
        in_specs=[
            pl.BlockSpec((256 // nblk, D), lambda i: (i, 0)),
            pl.BlockSpec((96, D), lambda i: (0, 0)),
        ],
        out_specs=pl.BlockSpec((256 // nblk, 96, D), lambda i: (i, 0, 0)),
        out_shape=jax.ShapeDtypeStruct((256, 96, D), jnp.float32),
    )(ptok, pdur)


def _idx_body(bar, pos, tok, dur, phr, i1_ref, i2_ref, i3_ref):
    r = lax.broadcasted_iota(jnp.int32, (400, 128), 0)
    c = lax.broadcasted_iota(jnp.int32, (400, 128), 1)
    l = lax.rem(r * 128 + c, L)
    i1_ref[...] = bar[...] * 32 + phr[...]
    i2_ref[...] = pos[...] * LPAD + l
    i3_ref[...] = tok[...] * 96 + dur[...]


def _fuse_idx(bar, pos, tok, dur, phr):
    return pl.pallas_call(
        _idx_body,
        out_shape=(
            jax.ShapeDtypeStruct((400, 128), jnp.int32),
            jax.ShapeDtypeStruct((400, 128), jnp.int32),
            jax.ShapeDtypeStruct((400, 128), jnp.int32),
        ),
    )(bar, pos, tok, dur, phr)


@functools.lru_cache(maxsize=None)
def _make_sc_gather():
    mesh = plsc.VectorSubcoreMesh(core_axis_name="c", subcore_axis_name="s")
    return functools.partial(
        pl.kernel,
        mesh=mesh,
        out_type=jax.ShapeDtypeStruct((NT, D), jnp.float32),
        scratch_types=[
            pltpu.VMEM((3, NCHUNK, T), jnp.int32),
            pltpu.VMEM((3 * T, D), jnp.float32),
            pltpu.VMEM((3 * T, D), jnp.float32),
            pltpu.SemaphoreType.DMA,
            pltpu.SemaphoreType.DMA,
            pltpu.SemaphoreType.DMA,
            pltpu.SemaphoreType.DMA,
        ],
    )(_sc_gather_body)


def _sc_gather_body(idx_h, t1_h, t2_h, t3_h, out_h, idxw, bufa, bufb, ga, gb, oa, ob):
    wid = lax.axis_index("s") * NC + lax.axis_index("c")
    base = wid * PER_W

    pltpu.sync_copy(idx_h.at[wid], idxw)

    def fire(c, buf, sem):
        pltpu.async_copy(t1_h.at[idxw.at[0, c]], buf.at[pl.ds(0, T)], sem)
        pltpu.async_copy(t2_h.at[idxw.at[1, c]], buf.at[pl.ds(T, T)], sem)
        pltpu.async_copy(t3_h.at[idxw.at[2, c]], buf.at[pl.ds(2 * T, T)], sem)

    def drain_gather(buf, sem):
        pltpu.make_async_copy(out_h.at[pl.ds(base, 3 * T)], buf, sem).wait()

    def reduce(buf):
        def row_add(r, carry):
            for cc in range(D // 16):
                sl = pl.ds(cc * 16, 16)
                buf[r, sl] = buf[r, sl] + buf[r + T, sl] + buf[r + 2 * T, sl]
            return carry
        lax.fori_loop(0, T, row_add, 0)

    def fire_out(c, buf, sem):
        pltpu.async_copy(buf.at[pl.ds(0, T)], out_h.at[pl.ds(base + c * T, T)], sem)

    def wait_out(buf, sem):
        pltpu.make_async_copy(out_h.at[pl.ds(base, T)], buf.at[pl.ds(0, T)], sem).wait()

    fire(0, bufa, ga)
    fire(1, bufb, gb)

    def pair(p, carry):
        c0 = 2 * p
        drain_gather(bufa, ga)
        reduce(bufa)
        fire_out(c0, bufa, oa)
        drain_gather(bufb, gb)

        @pl.when(p < NPAIR - 1)
        def _():
            wait_out(bufa, oa)
            fire(c0 + 2, bufa, ga)

        reduce(bufb)
        fire_out(c0 + 1, bufb, ob)

        @pl.when(p < NPAIR - 1)
        def _():
            wait_out(bufb, ob)
            fire(c0 + 3, bufb, gb)

        return carry

    lax.fori_loop(0, NPAIR, pair, 0)
    wait_out(bufa, oa)
    wait_out(bufb, ob)


def kernel(bar, pos, token, dur, phrase, W_bar, W_pos, W_token, W_dur, W_phrase, W_proj, b_proj):
    pe = _pos_enc(L, D)
    t1_3d, t2_3d, ptok, pdur = _prep(W_bar, W_pos, W_token, W_dur, W_phrase, W_proj, b_proj, pe)
    t3_3d = _t3(ptok, pdur)
    t1 = t1_3d.reshape(N_T1, D)
    t2 = t2_3d.reshape(N_T2, D)
    t3 = t3_3d.reshape(N_T3, D)

    flat2 = lambda x: jnp.asarray(x, jnp.int32).reshape(400, 128)
    i1, i2, i3 = _fuse_idx(flat2(bar), flat2(pos), flat2(token), flat2(dur), flat2(phrase))
    idx = jnp.stack([i.reshape(NW, PER_W) for i in (i1, i2, i3)], axis=1)
    idx = idx.reshape(NW, 3, NCHUNK, T)

    out = _make_sc_gather()(idx, t1, t2, t3)
    return out.reshape(B, L, D)
